# baseline (speedup 1.0000x reference)
import functools

import jax
import jax.numpy as jnp
from jax import lax
from jax.experimental import pallas as pl
from jax.experimental.pallas import tpu as pltpu
from jax.experimental.pallas import tpu_sc as plsc

_NC = 2
_NS = 16
_NW = _NC * _NS
_L = 16


@functools.partial(jax.jit, static_argnames=("chunk",))
def _sc_embed(x, table, pe, *, chunk):
    b, s = x.shape
    d = table.shape[1]
    s_per_w = s // _NW
    n_chunks = s_per_w // chunk
    n_iter = n_chunks * b
    mesh = plsc.VectorSubcoreMesh(core_axis_name="c", subcore_axis_name="s")

    @functools.partial(
        pl.kernel,
        mesh=mesh,
        out_type=jax.ShapeDtypeStruct((b, s, d), jnp.float32),
        scratch_types=[
            pltpu.VMEM((b * s_per_w,), jnp.int32),
            pltpu.VMEM((chunk, d), jnp.float32),
            pltpu.VMEM((chunk, d), jnp.float32),
            pltpu.VMEM((chunk, d), jnp.float32),
            pltpu.VMEM((chunk, d), jnp.float32),
            pltpu.VMEM((chunk, d), jnp.float32),
            pltpu.SemaphoreType.DMA,
            pltpu.SemaphoreType.DMA,
            pltpu.SemaphoreType.DMA,
            pltpu.SemaphoreType.DMA,
            pltpu.SemaphoreType.DMA,
            pltpu.SemaphoreType.DMA,
            pltpu.SemaphoreType.DMA,
            pltpu.SemaphoreType.DMA,
        ],
    )
    def k(x_hbm, tab_hbm, pe_hbm, out_hbm,
          idx_v, rows0, rows1, rows2, pe0, pe1,
          g0, g1, g2, o0, o1, o2, psem, isem):
        rows = (rows0, rows1, rows2)
        pes = (pe0, pe1)
        gsems = (g0, g1, g2)
        osems = (o0, o1, o2)

        wid = lax.axis_index("s") * _NC + lax.axis_index("c")
        s_base = wid * s_per_w

        idescs = [
            pltpu.async_copy(x_hbm.at[bi, pl.ds(s_base, s_per_w)],
                             idx_v.at[pl.ds(bi * s_per_w, s_per_w)], isem)
            for bi in range(b)
        ]
        pedesc0 = pltpu.async_copy(pe_hbm.at[0, pl.ds(s_base, chunk)],
                                   pe0, psem)

        def idx_view(c, bi):
            return idx_v.at[pl.ds(bi * s_per_w + c * chunk, chunk)]

        def out_view(c, bi):
            return out_hbm.at[bi, pl.ds(s_base + c * chunk, chunk)]

        gdesc = [None, None, None]
        odesc = [None, None, None]
        pedesc = None
        idescs[0].wait()
        gdesc[0] = pltpu.async_copy(tab_hbm.at[idx_view(0, 0)], rows0, g0)
        for dsc in idescs[1:]:
            dsc.wait()
        gdesc[1] = pltpu.async_copy(tab_hbm.at[idx_view(0, 1)], rows1, g1)

        for i in range(n_iter):
            p = i % 3
            c, bi = divmod(i, b)
            if bi == 0 and c + 1 < n_chunks:
                pedesc = pltpu.async_copy(
                    pe_hbm.at[0, pl.ds(s_base + (c + 1) * chunk, chunk)],
                    pes[(c + 1) % 2], psem)
            if i == 0:
                pedesc0.wait()
            if bi == 0 and c > 0:
                pedesc.wait()
            gdesc[p].wait()
            if i + 2 < n_iter:
                q = (i + 2) % 3
                if odesc[q] is not None:
                    odesc[q].wait()
                cn, bn = divmod(i + 2, b)
                gdesc[q] = pltpu.async_copy(
                    tab_hbm.at[idx_view(cn, bn)], rows[q], gsems[q])

            pe_buf = pes[c % 2]
            rows_buf = rows[p]

            @plsc.parallel_loop(0, chunk // 2)
            def add_row(r2):
                for rr in range(2):
                    r = r2 * 2 + rr
                    for j in range(d // _L):
                        sl = pl.ds(j * _L, _L)
                        plsc.addupdate(rows_buf.at[r, sl], pe_buf[r, sl])
            odesc[p] = pltpu.async_copy(rows_buf, out_view(c, bi), osems[p])

        odesc[0].wait()
        odesc[1].wait()
        odesc[2].wait()

    return k(x, table, pe)


def kernel(x, table, pe):
    return _sc_embed(x.astype(jnp.int32), table, pe, chunk=32)


# speedup vs baseline: 1.0499x; 1.0499x over previous
import functools

import jax
import jax.numpy as jnp
from jax import lax
from jax.experimental import pallas as pl
from jax.experimental.pallas import tpu as pltpu
from jax.experimental.pallas import tpu_sc as plsc

_NC = 2
_NS = 16
_NW = _NC * _NS
_L = 16


@functools.partial(jax.jit, static_argnames=("chunk",))
def _sc_embed(x, table, pe, *, chunk):
    b, s = x.shape
    d = table.shape[1]
    s_per_w = s // _NW
    n_chunks = s_per_w // chunk
    n_iter = n_chunks * b
    mesh = plsc.VectorSubcoreMesh(core_axis_name="c", subcore_axis_name="s")

    @functools.partial(
        pl.kernel,
        mesh=mesh,
        out_type=jax.ShapeDtypeStruct((b, s, d), jnp.float32),
        scratch_types=[
            pltpu.VMEM((b * s_per_w,), jnp.int32),
            pltpu.VMEM((chunk, d), jnp.float32),
            pltpu.VMEM((chunk, d), jnp.float32),
            pltpu.VMEM((chunk, d), jnp.float32),
            pltpu.VMEM((chunk, d), jnp.float32),
            pltpu.VMEM((chunk, d), jnp.float32),
            pltpu.SemaphoreType.DMA,
            pltpu.SemaphoreType.DMA,
            pltpu.SemaphoreType.DMA,
            pltpu.SemaphoreType.DMA,
            pltpu.SemaphoreType.DMA,
            pltpu.SemaphoreType.DMA,
            pltpu.SemaphoreType.DMA,
            pltpu.SemaphoreType.DMA,
        ],
    )
    def k(x_hbm, tab_hbm, pe_hbm, out_hbm,
          idx_v, rows0, rows1, rows2, pe0, pe1,
          g0, g1, g2, o0, o1, o2, psem, isem):
        rows = (rows0, rows1, rows2)
        pes = (pe0, pe1)
        gsems = (g0, g1, g2)
        osems = (o0, o1, o2)

        wid = lax.axis_index("s") * _NC + lax.axis_index("c")
        s_base = wid * s_per_w

        idescs = [
            pltpu.async_copy(x_hbm.at[bi, pl.ds(s_base, s_per_w)],
                             idx_v.at[pl.ds(bi * s_per_w, s_per_w)], isem)
            for bi in range(b)
        ]
        pedesc0 = pltpu.async_copy(pe_hbm.at[0, pl.ds(s_base, chunk)],
                                   pe0, psem)

        def idx_view(c, bi):
            return idx_v.at[pl.ds(bi * s_per_w + c * chunk, chunk)]

        def out_view(c, bi):
            return out_hbm.at[bi, pl.ds(s_base + c * chunk, chunk)]

        gdesc = [None, None, None]
        odesc = [None, None, None]
        pedesc = None
        idescs[0].wait()
        gdesc[0] = pltpu.async_copy(tab_hbm.at[idx_view(0, 0)], rows0, g0)
        for dsc in idescs[1:]:
            dsc.wait()
        gdesc[1] = pltpu.async_copy(tab_hbm.at[idx_view(0, 1)], rows1, g1)

        for i in range(n_iter):
            p = i % 3
            c, bi = divmod(i, b)
            if bi == 0 and c + 1 < n_chunks:
                pedesc = pltpu.async_copy(
                    pe_hbm.at[0, pl.ds(s_base + (c + 1) * chunk, chunk)],
                    pes[(c + 1) % 2], psem)
            if i == 0:
                pedesc0.wait()
            if bi == 0 and c > 0:
                pedesc.wait()
            gdesc[p].wait()
            if i + 2 < n_iter:
                q = (i + 2) % 3
                if odesc[q] is not None:
                    odesc[q].wait()
                cn, bn = divmod(i + 2, b)
                gdesc[q] = pltpu.async_copy(
                    tab_hbm.at[idx_view(cn, bn)], rows[q], gsems[q])

            pe_buf = pes[c % 2]
            rows_buf = rows[p]

            def add_row(r2, carry):
                for rr in range(2):
                    r = r2 * 2 + rr
                    for j in range(d // _L):
                        sl = pl.ds(j * _L, _L)
                        plsc.addupdate(rows_buf.at[r, sl], pe_buf[r, sl])
                return carry

            lax.fori_loop(0, chunk // 2, add_row, 0)
            odesc[p] = pltpu.async_copy(rows_buf, out_view(c, bi), osems[p])

        odesc[0].wait()
        odesc[1].wait()
        odesc[2].wait()

    return k(x, table, pe)


def kernel(x, table, pe):
    return _sc_embed(x.astype(jnp.int32), table, pe, chunk=32)
